# baseline (speedup 1.0000x reference)
import functools

import jax
import jax.numpy as jnp
from jax import lax
from jax.experimental import pallas as pl
from jax.experimental.pallas import tpu as pltpu
from jax.experimental.pallas import tpu_sc as plsc

BATCH = 16384
NC, NS = 1, 16
NW = NC * NS
IPW = BATCH // NW
NCH = 4
CH = IPW // NCH

_MESH = plsc.VectorSubcoreMesh(core_axis_name="c", subcore_axis_name="s",
                               num_cores=NC)


@functools.partial(
    pl.kernel,
    out_type=jax.ShapeDtypeStruct((BATCH,), jnp.float32),
    mesh=_MESH,
    scratch_types=(
        [pltpu.VMEM((IPW,), jnp.int32), pltpu.VMEM((IPW,), jnp.float32)]
        + [pltpu.SemaphoreType.DMA] * (2 * NCH)
    ),
)
def _sc_gather(w_hbm, idx_hbm, out_hbm, idx_v, val_v, *sems):
    isem, gsem = sems[:NCH], sems[NCH:]
    wid = lax.axis_index("s") * NC + lax.axis_index("c")
    base = wid * IPW
    loads = [
        pltpu.async_copy(idx_hbm.at[pl.ds(base + k * CH, CH)],
                         idx_v.at[pl.ds(k * CH, CH)], isem[k])
        for k in range(NCH)
    ]
    gathers = []
    for k in range(NCH):
        loads[k].wait()
        gathers.append(
            pltpu.async_copy(w_hbm.at[idx_v.at[pl.ds(k * CH, CH)]],
                             val_v.at[pl.ds(k * CH, CH)], gsem[k]))
    stores = []
    for k in range(NCH):
        gathers[k].wait()
        stores.append(
            pltpu.async_copy(val_v.at[pl.ds(k * CH, CH)],
                             out_hbm.at[pl.ds(base + k * CH, CH)], isem[k]))
    for s in stores:
        s.wait()


def kernel(weights, indices):
    return _sc_gather(weights, indices.astype(jnp.int32))
